# Initial kernel scaffold; baseline (speedup 1.0000x reference)
import jax
import jax.numpy as jnp
from jax.experimental import pallas as pl


def kernel(x, mask, edge_index, W1, b1, W2, b2):
    raise NotImplementedError("write your pallas kernel here")


# speedup vs baseline: 12.3070x; 12.3070x over previous
import functools

import jax
import jax.numpy as jnp
from jax import lax
from jax.experimental import pallas as pl
from jax.experimental.pallas import tpu as pltpu
from jax.experimental.pallas import tpu_sc as plsc

NUM_NODES = 10000
NUM_FEAT = 128
CHUNK = 80
NS = 16
NC = 2
NODES_PER_TILE = 624
NODES_TAIL = NUM_NODES - NS * NODES_PER_TILE
NODE_BLOCK = 1000


def _tile_rows(sid):
    return pl.multiple_of(sid * NODES_PER_TILE, 8)


_sc_mesh = plsc.VectorSubcoreMesh(core_axis_name="c", subcore_axis_name="s")
_sc_params = pltpu.CompilerParams(use_tc_tiling_on_sc=False)


def _deg_body(dst_hbm, o0, o1, idx_cur, ones_v, zero_v, acc, dsem):
    cid = lax.axis_index("c")
    sid = lax.axis_index("s")
    wid = cid * NS + sid
    rows_per_tile = dst_hbm.shape[0] // (NC * NS * CHUNK)

    def fill_ones(i, c):
        ones_v[i] = jnp.full((16,), 1.0, jnp.float32)
        return c

    lax.fori_loop(0, CHUNK, fill_ones, 0)

    def fill_zero(i, c):
        zero_v[i] = jnp.zeros((16,), jnp.float32)
        return c

    lax.fori_loop(0, NODES_PER_TILE, fill_zero, 0)

    nbase = _tile_rows(sid)

    def zero_via(o_ref):
        pltpu.sync_copy(zero_v, o_ref.at[pl.ds(nbase, NODES_PER_TILE)])
        pltpu.sync_copy(o_ref.at[pl.ds(nbase, NODES_PER_TILE)],
                        acc.at[pl.ds(nbase, NODES_PER_TILE)])

        @pl.when(sid == NS - 1)
        def _():
            pltpu.sync_copy(zero_v.at[pl.ds(0, NODES_TAIL)],
                            o_ref.at[pl.ds(NS * NODES_PER_TILE, NODES_TAIL)])
            pltpu.sync_copy(o_ref.at[pl.ds(NS * NODES_PER_TILE, NODES_TAIL)],
                            acc.at[pl.ds(NS * NODES_PER_TILE, NODES_TAIL)])

    @pl.when(cid == 0)
    def _():
        zero_via(o0)

    @pl.when(cid == 1)
    def _():
        zero_via(o1)

    edges_per_tile = dst_hbm.shape[0] // (NC * NS)
    ebase = wid * edges_per_tile
    plsc.subcore_barrier()

    def body(c, carry):
        off = pl.multiple_of(ebase + c * CHUNK, 8)
        pltpu.sync_copy(dst_hbm.at[pl.ds(off, CHUNK)], idx_cur)
        pltpu.sync_copy(ones_v, acc.at[idx_cur], add=True)
        return carry

    lax.fori_loop(0, rows_per_tile, body, 0)
    plsc.subcore_barrier()

    def writeback(o_ref):
        pltpu.sync_copy(acc.at[pl.ds(nbase, NODES_PER_TILE)],
                        o_ref.at[pl.ds(nbase, NODES_PER_TILE)])

        @pl.when(sid == NS - 1)
        def _():
            pltpu.sync_copy(acc.at[pl.ds(NS * NODES_PER_TILE, NODES_TAIL)],
                            o_ref.at[pl.ds(NS * NODES_PER_TILE, NODES_TAIL)])

    @pl.when(cid == 0)
    def _():
        writeback(o0)

    @pl.when(cid == 1)
    def _():
        writeback(o1)


@jax.jit
def _deg_call(dst1):
    n_edges = dst1.shape[0]
    return pl.kernel(
        _deg_body,
        out_type=(
            jax.ShapeDtypeStruct((NUM_NODES, 16), jnp.float32),
            jax.ShapeDtypeStruct((NUM_NODES, 16), jnp.float32),
        ),
        mesh=_sc_mesh,
        compiler_params=_sc_params,
        scratch_types=[
            pltpu.VMEM((CHUNK,), jnp.int32),
            pltpu.VMEM((CHUNK, 16), jnp.float32),
            pltpu.VMEM((NODES_PER_TILE, 16), jnp.float32),
            pltpu.VMEM_SHARED((NUM_NODES, 16), jnp.float32),
            pltpu.SemaphoreType.DMA,
        ],
    )(dst1)


def _prop_body(h0, h1, h2, h3, src_hbm, dst_hbm, o0, o1, o2, o3,
               src_v, dst_v, rows0, rows1, acc, gsem, ssem):
    cid = lax.axis_index("c")
    sid = lax.axis_index("s")
    edges_per_tile = src_hbm.shape[0] // NS
    sec_edges = src_v.shape[0]
    n_secs = edges_per_tile // sec_edges
    n_pairs = sec_edges // (2 * CHUNK)
    nbase = _tile_rows(sid)
    ebase = sid * edges_per_tile

    def do_hour(h_ref, o_ref):
        pltpu.sync_copy(h_ref.at[pl.ds(nbase, NODES_PER_TILE)],
                        acc.at[pl.ds(nbase, NODES_PER_TILE)])

        @pl.when(sid == NS - 1)
        def _():
            pltpu.sync_copy(h_ref.at[pl.ds(NS * NODES_PER_TILE, NODES_TAIL)],
                            acc.at[pl.ds(NS * NODES_PER_TILE, NODES_TAIL)])

        plsc.subcore_barrier()

        def gather(off, rows_b):
            pltpu.async_copy(h_ref.at[src_v.at[pl.ds(off, CHUNK)]],
                             rows_b, gsem)

        def scatter(off, rows_b):
            pltpu.async_copy(rows_b, acc.at[dst_v.at[pl.ds(off, CHUNK)]],
                             ssem, add=True)

        def wait_g():
            pltpu.make_async_copy(h_ref.at[pl.ds(0, CHUNK)], rows0, gsem).wait()

        def wait_s():
            pltpu.make_async_copy(h_ref.at[pl.ds(0, CHUNK)], rows0, ssem).wait()

        def section(s, carry):
            sbase = pl.multiple_of(ebase + s * sec_edges, 8)
            pltpu.sync_copy(src_hbm.at[pl.ds(sbase, sec_edges)], src_v)
            pltpu.sync_copy(dst_hbm.at[pl.ds(sbase, sec_edges)], dst_v)
            gather(0, rows0)

            def body(p, carry):
                off0 = pl.multiple_of(p * (2 * CHUNK), 8)
                off1 = pl.multiple_of(off0 + CHUNK, 8)
                off2 = jnp.where(p == n_pairs - 1, 0, off0 + 2 * CHUNK)
                off2 = pl.multiple_of(off2, 8)

                wait_g()

                @pl.when(p > 0)
                def _():
                    wait_s()

                gather(off1, rows1)
                scatter(off0, rows0)
                wait_g()
                wait_s()
                gather(off2, rows0)
                scatter(off1, rows1)
                return carry

            lax.fori_loop(0, n_pairs, body, 0)
            wait_g()
            wait_s()
            return carry

        lax.fori_loop(0, n_secs, section, 0)
        plsc.subcore_barrier()
        pltpu.sync_copy(acc.at[pl.ds(nbase, NODES_PER_TILE)],
                        o_ref.at[pl.ds(nbase, NODES_PER_TILE)])

        @pl.when(sid == NS - 1)
        def _():
            pltpu.sync_copy(acc.at[pl.ds(NS * NODES_PER_TILE, NODES_TAIL)],
                            o_ref.at[pl.ds(NS * NODES_PER_TILE, NODES_TAIL)])

        plsc.subcore_barrier()

    @pl.when(cid == 0)
    def _():
        do_hour(h0, o0)
        do_hour(h1, o1)

    @pl.when(cid == 1)
    def _():
        do_hour(h2, o2)
        do_hour(h3, o3)


@jax.jit
def _prop_call(h4, src1, dst1):
    n_edges = src1.shape[0]
    node_t = jax.ShapeDtypeStruct((NUM_NODES, NUM_FEAT), jnp.float32)
    outs = pl.kernel(
        _prop_body,
        out_type=(node_t,) * 4,
        mesh=_sc_mesh,
        compiler_params=_sc_params,
        scratch_types=[
            pltpu.VMEM((4000,), jnp.int32),
            pltpu.VMEM((4000,), jnp.int32),
            pltpu.VMEM((CHUNK, NUM_FEAT), jnp.float32),
            pltpu.VMEM((CHUNK, NUM_FEAT), jnp.float32),
            pltpu.VMEM_SHARED((NUM_NODES, NUM_FEAT), jnp.float32),
            pltpu.SemaphoreType.DMA,
            pltpu.SemaphoreType.DMA,
        ],
    )(h4[0], h4[1], h4[2], h4[3], src1, dst1)
    return jnp.stack(outs, axis=0)


def _dinv_block(d0_ref, d1_ref):
    deg = d0_ref[:, :1] + d1_ref[:, :1] + 1.0
    return lax.rsqrt(deg)


def _mm1_kernel(x_ref, mask_ref, w1b_ref, d0_ref, d1_ref, w_ref, o_ref):
    dinv = _dinv_block(d0_ref, d1_ref)
    h = jnp.dot(x_ref[0], w_ref[...], preferred_element_type=jnp.float32)
    m = mask_ref[pl.ds(pl.program_id(0), 1), :]
    o_ref[0] = (h + m * w1b_ref[...]) * dinv


@jax.jit
def _mm1_call(xh, mask41, w1b, d0, d1, w1a):
    grid = (4, NUM_NODES // NODE_BLOCK)
    return pl.pallas_call(
        _mm1_kernel,
        grid=grid,
        in_specs=[
            pl.BlockSpec((1, NODE_BLOCK, NUM_FEAT), lambda h, n: (h, n, 0)),
            pl.BlockSpec((4, 1), lambda h, n: (0, 0)),
            pl.BlockSpec((1, NUM_FEAT), lambda h, n: (0, 0)),
            pl.BlockSpec((NODE_BLOCK, 16), lambda h, n: (n, 0)),
            pl.BlockSpec((NODE_BLOCK, 16), lambda h, n: (n, 0)),
            pl.BlockSpec((NUM_FEAT, NUM_FEAT), lambda h, n: (0, 0)),
        ],
        out_specs=pl.BlockSpec((1, NODE_BLOCK, NUM_FEAT), lambda h, n: (h, n, 0)),
        out_shape=jax.ShapeDtypeStruct((4, NUM_NODES, NUM_FEAT), jnp.float32),
    )(xh, mask41, w1b, d0, d1, w1a)


def _mm2_kernel(s_ref, b1_ref, d0_ref, d1_ref, w_ref, o_ref):
    dinv = _dinv_block(d0_ref, d1_ref)
    enc = s_ref[0] * dinv + b1_ref[...]
    o_ref[0] = jnp.dot(enc, w_ref[...], preferred_element_type=jnp.float32) * dinv


@jax.jit
def _mm2_call(s1, b1r, d0, d1, w2):
    grid = (4, NUM_NODES // NODE_BLOCK)
    return pl.pallas_call(
        _mm2_kernel,
        grid=grid,
        in_specs=[
            pl.BlockSpec((1, NODE_BLOCK, NUM_FEAT), lambda h, n: (h, n, 0)),
            pl.BlockSpec((1, NUM_FEAT), lambda h, n: (0, 0)),
            pl.BlockSpec((NODE_BLOCK, 16), lambda h, n: (n, 0)),
            pl.BlockSpec((NODE_BLOCK, 16), lambda h, n: (n, 0)),
            pl.BlockSpec((NUM_FEAT, NUM_FEAT), lambda h, n: (0, 0)),
        ],
        out_specs=pl.BlockSpec((1, NODE_BLOCK, NUM_FEAT), lambda h, n: (h, n, 0)),
        out_shape=jax.ShapeDtypeStruct((4, NUM_NODES, NUM_FEAT), jnp.float32),
    )(s1, b1r, d0, d1, w2)


def _fin_kernel(s_ref, b2_ref, d0_ref, d1_ref, o_ref):
    dinv = _dinv_block(d0_ref, d1_ref)
    o_ref[0] = jnp.tanh(s_ref[0] * dinv + b2_ref[...])


@jax.jit
def _fin_call(s2, b2r, d0, d1):
    grid = (4, NUM_NODES // NODE_BLOCK)
    return pl.pallas_call(
        _fin_kernel,
        grid=grid,
        in_specs=[
            pl.BlockSpec((1, NODE_BLOCK, NUM_FEAT), lambda h, n: (h, n, 0)),
            pl.BlockSpec((1, NUM_FEAT), lambda h, n: (0, 0)),
            pl.BlockSpec((NODE_BLOCK, 16), lambda h, n: (n, 0)),
            pl.BlockSpec((NODE_BLOCK, 16), lambda h, n: (n, 0)),
        ],
        out_specs=pl.BlockSpec((1, NODE_BLOCK, NUM_FEAT), lambda h, n: (h, n, 0)),
        out_shape=jax.ShapeDtypeStruct((4, NUM_NODES, NUM_FEAT), jnp.float32),
    )(s2, b2r, d0, d1)


def kernel(x, mask, edge_index, W1, b1, W2, b2):
    B, H, N, F = x.shape
    xh = x[0]
    src1 = edge_index[0].astype(jnp.int32)
    dst1 = edge_index[1].astype(jnp.int32)

    d0, d1 = _deg_call(dst1)

    w1a = W1[:F]
    w1b = W1[F].reshape(1, F)
    mask41 = mask[0].reshape(H, 1)

    h1 = _mm1_call(xh, mask41, w1b, d0, d1, w1a)
    s1 = _prop_call(h1, src1, dst1)
    h2 = _mm2_call(s1, b1.reshape(1, F), d0, d1, W2)
    s2 = _prop_call(h2, src1, dst1)
    out = _fin_call(s2, b2.reshape(1, F), d0, d1)
    return out[None]
